# baseline (device time: 32549 ns/iter reference)
import jax
import jax.numpy as jnp
from jax import lax
from jax.experimental import pallas as pl
from jax.experimental.pallas import tpu as pltpu

N_DEV = 4
B = 2
S = 256
HQ = 4
DH = 64
HD = HQ * DH
KVW = 2 * B * HD
NEG = -1e9
HALF = S // 2
G = 32

_sem_signal = getattr(pl, "semaphore_signal", None) or pltpu.semaphore_signal
_sem_wait = getattr(pl, "semaphore_wait", None) or pltpu.semaphore_wait
_DeviceIdType = getattr(pl, "DeviceIdType", None) or pltpu.DeviceIdType


def _body(x_ref, wq_ref, k_ref, v_ref, wo_ref, out_ref,
          kv_s, kv_full, far32, q32_buf, st_own, st_stage, st_in, ss, rs):
    my_pos = lax.axis_index("i")
    right = lax.rem(my_pos + 1, N_DEV)
    left = lax.rem(my_pos + N_DEV - 1, N_DEV)

    for b in range(B):
        kv_s[:, b * 2 * HD:b * 2 * HD + HD] = k_ref[b].reshape(S, HD)
        kv_s[:, b * 2 * HD + HD:(b + 1) * 2 * HD] = v_ref[b].reshape(S, HD)

    barrier_sem = pltpu.get_barrier_semaphore()
    for nbr in (left, right):
        _sem_signal(barrier_sem, inc=1, device_id=(nbr,),
                    device_id_type=_DeviceIdType.MESH)
    _sem_wait(barrier_sem, 2)

    dA_ltop = pltpu.make_async_remote_copy(
        src_ref=kv_s.at[pl.ds(0, HALF)], dst_ref=kv_full.at[0, pl.ds(0, HALF)],
        send_sem=ss.at[0], recv_sem=rs.at[0],
        device_id=(left,), device_id_type=_DeviceIdType.MESH)
    dA_rbot = pltpu.make_async_remote_copy(
        src_ref=kv_s.at[pl.ds(HALF, HALF)],
        dst_ref=kv_full.at[1, pl.ds(HALF, HALF)],
        send_sem=ss.at[1], recv_sem=rs.at[1],
        device_id=(right,), device_id_type=_DeviceIdType.MESH)
    dA_r32 = pltpu.make_async_remote_copy(
        src_ref=kv_s.at[pl.ds(0, G)], dst_ref=kv_full.at[1, pl.ds(0, G)],
        send_sem=ss.at[2], recv_sem=rs.at[2],
        device_id=(right,), device_id_type=_DeviceIdType.MESH)
    dA_ltop.start()
    dA_rbot.start()
    dA_r32.start()

    q = [jnp.dot(x_ref[b], wq_ref[...],
                 preferred_element_type=jnp.float32) * 0.125
         for b in range(B)]

    @pl.when(my_pos == 0)
    def _():
        for b in range(B):
            q32_buf[b * G:(b + 1) * G, :] = q[b][0:G, :]
        for dev, sidx in ((right, 4), (left, 5)):
            d = pltpu.make_async_remote_copy(
                src_ref=q32_buf, dst_ref=q32_buf,
                send_sem=ss.at[sidx], recv_sem=rs.at[4],
                device_id=(dev,), device_id_type=_DeviceIdType.MESH)
            d.start()
            d.wait_send()

    @pl.when((my_pos == 1) | (my_pos == 3))
    def _():
        recv = pltpu.make_async_remote_copy(
            src_ref=q32_buf, dst_ref=q32_buf,
            send_sem=ss.at[7], recv_sem=rs.at[4],
            device_id=(left,), device_id_type=_DeviceIdType.MESH)
        recv.wait_recv()

    @pl.when(my_pos == 3)
    def _():
        fwd = pltpu.make_async_remote_copy(
            src_ref=q32_buf, dst_ref=q32_buf,
            send_sem=ss.at[4], recv_sem=rs.at[4],
            device_id=(left,), device_id_type=_DeviceIdType.MESH)
        fwd.start()
        fwd.wait_send()

    @pl.when(my_pos == 2)
    def _():
        recv = pltpu.make_async_remote_copy(
            src_ref=q32_buf, dst_ref=q32_buf,
            send_sem=ss.at[7], recv_sem=rs.at[4],
            device_id=(right,), device_id_type=_DeviceIdType.MESH)
        recv.wait_recv()

    dn = (((1,), (1,)), ((), ()))
    for b in range(B):
        for h in range(HQ):
            c0 = b * 2 * HD + h * DH
            cv = c0 + HD
            q32 = q32_buf[b * G:(b + 1) * G, h * DH:(h + 1) * DH]
            s = lax.dot_general(q32, kv_s[:, c0:c0 + DH], dn,
                                preferred_element_type=jnp.float32)
            m = jnp.max(s, axis=1, keepdims=True)
            w = jnp.exp(s - m)
            l = jnp.sum(w, axis=1, keepdims=True)
            acc = jnp.dot(w, kv_s[:, cv:cv + DH],
                          preferred_element_type=jnp.float32)
            r0 = (b * HQ + h) * G
            st_own[r0:r0 + G, 0:DH] = acc
            st_own[r0:r0 + G, DH:DH + 1] = m
            st_own[r0:r0 + G, DH + 1:DH + 2] = l

    @pl.when(my_pos == 1)
    def _():
        d = pltpu.make_async_remote_copy(
            src_ref=st_own, dst_ref=st_in.at[0],
            send_sem=ss.at[4], recv_sem=rs.at[5],
            device_id=(left,), device_id_type=_DeviceIdType.MESH)
        d.start()
        d.wait_send()

    @pl.when(my_pos == 2)
    def _():
        d = pltpu.make_async_remote_copy(
            src_ref=st_own, dst_ref=st_stage,
            send_sem=ss.at[4], recv_sem=rs.at[5],
            device_id=(right,), device_id_type=_DeviceIdType.MESH)
        d.start()
        d.wait_send()

    @pl.when(my_pos == 3)
    def _():
        d = pltpu.make_async_remote_copy(
            src_ref=st_own, dst_ref=st_in.at[1],
            send_sem=ss.at[5], recv_sem=rs.at[6],
            device_id=(right,), device_id_type=_DeviceIdType.MESH)
        d.start()
        d.wait_send()

    dA_ltop.wait_recv()
    dB = pltpu.make_async_remote_copy(
        src_ref=kv_full.at[0, pl.ds(0, G)], dst_ref=far32,
        send_sem=ss.at[3], recv_sem=rs.at[3],
        device_id=(left,), device_id_type=_DeviceIdType.MESH)
    dB.start()
    dA_rbot.wait_recv()

    qi = my_pos * S + lax.broadcasted_iota(jnp.int32, (S, 1), 0)
    off_own = my_pos * S
    off_R = lax.rem(my_pos + 1, N_DEV) * S
    off_F = lax.rem(my_pos + 2, N_DEV) * S
    off_L = lax.rem(my_pos + 3, N_DEV) * S

    def seg_mask(off, r0, n):
        ki = off + r0 + lax.broadcasted_iota(jnp.int32, (S, n), 1)
        return (jnp.abs(qi - ki) <= 128) | (ki < G) | (qi < G)

    mask1 = jnp.concatenate(
        [seg_mask(off_L, HALF, HALF), seg_mask(off_own, 0, S),
         seg_mask(off_R, 0, HALF)], axis=1)
    mask2 = seg_mask(off_L, 0, G)
    mask3 = seg_mask(off_F, 0, G)

    segs1 = [(kv_full, 1, HALF, HALF), (kv_s, None, 0, S),
             (kv_full, 0, 0, HALF)]

    def seg_k(ref, slot, r0, n, c0):
        blk = ref[...] if slot is None else ref[slot]
        return blk[r0:r0 + n, c0:c0 + DH]

    part = {}
    for b in range(B):
        for h in range(HQ):
            c0 = b * 2 * HD + h * DH
            q_bh = q[b][:, h * DH:(h + 1) * DH]
            s1 = jnp.concatenate(
                [lax.dot_general(q_bh, seg_k(r, sl, r0, n, c0), dn,
                                 preferred_element_type=jnp.float32)
                 for (r, sl, r0, n) in segs1], axis=1)
            s1 = jnp.where(mask1, s1, NEG)
            m1 = jnp.max(s1, axis=1, keepdims=True)
            w1 = jnp.exp(s1 - m1)
            l1 = jnp.sum(w1, axis=1, keepdims=True)
            acc = jnp.zeros((S, DH), jnp.float32)
            col = 0
            for (r, sl, r0, n) in segs1:
                acc = acc + jnp.dot(
                    w1[:, col:col + n], seg_k(r, sl, r0, n, c0 + HD),
                    preferred_element_type=jnp.float32)
                col += n
            part[(b, h)] = (m1, l1, acc)

    def upd(state, s_new, v_keys):
        m, l, acc = state
        mn = jnp.maximum(m, jnp.max(s_new, axis=1, keepdims=True))
        a = jnp.exp(m - mn)
        w = jnp.exp(s_new - mn)
        return (mn, l * a + jnp.sum(w, axis=1, keepdims=True),
                acc * a + jnp.dot(w, v_keys,
                                  preferred_element_type=jnp.float32))

    dA_r32.wait_recv()
    for b in range(B):
        for h in range(HQ):
            c0 = b * 2 * HD + h * DH
            q_bh = q[b][:, h * DH:(h + 1) * DH]
            s2 = lax.dot_general(q_bh, kv_full[1][0:G, c0:c0 + DH], dn,
                                 preferred_element_type=jnp.float32)
            s2 = jnp.where(mask2, s2, NEG)
            part[(b, h)] = upd(part[(b, h)], s2,
                               kv_full[1][0:G, c0 + HD:c0 + HD + DH])

    dB.wait_recv()
    for b in range(B):
        for h in range(HQ):
            c0 = b * 2 * HD + h * DH
            q_bh = q[b][:, h * DH:(h + 1) * DH]
            s3 = lax.dot_general(q_bh, far32[:, c0:c0 + DH], dn,
                                 preferred_element_type=jnp.float32)
            s3 = jnp.where(mask3, s3, NEG)
            part[(b, h)] = upd(part[(b, h)], s3,
                               far32[:, c0 + HD:c0 + HD + DH])

    @pl.when(my_pos == 3)
    def _():
        recv2 = pltpu.make_async_remote_copy(
            src_ref=st_stage, dst_ref=st_stage,
            send_sem=ss.at[7], recv_sem=rs.at[5],
            device_id=(left,), device_id_type=_DeviceIdType.MESH)
        recv2.wait_recv()
        fwd = pltpu.make_async_remote_copy(
            src_ref=st_stage, dst_ref=st_in.at[2],
            send_sem=ss.at[6], recv_sem=rs.at[7],
            device_id=(right,), device_id_type=_DeviceIdType.MESH)
        fwd.start()
        fwd.wait_send()

    @pl.when(my_pos == 0)
    def _():
        for ridx in (5, 6, 7):
            w = pltpu.make_async_remote_copy(
                src_ref=st_own, dst_ref=st_in.at[ridx - 5],
                send_sem=ss.at[7], recv_sem=rs.at[ridx],
                device_id=(right,), device_id_type=_DeviceIdType.MESH)
            w.wait_recv()

    is0 = my_pos == 0
    row_lt_g = lax.broadcasted_iota(jnp.int32, (S, 1), 0) < G
    for b in range(B):
        ctx_parts = []
        for h in range(HQ):
            m1, l1, acc = part[(b, h)]
            ctx_bh = acc / l1
            r0 = (b * HQ + h) * G
            srcs = [st_own] + [st_in.at[j] for j in range(3)]
            ms = [src[r0:r0 + G, DH:DH + 1] for src in srcs]
            mstar = ms[0]
            for mj in ms[1:]:
                mstar = jnp.maximum(mstar, mj)
            num = jnp.zeros((G, DH), jnp.float32)
            den = jnp.zeros((G, 1), jnp.float32)
            for src, mj in zip(srcs, ms):
                e = jnp.exp(mj - mstar)
                num = num + src[r0:r0 + G, 0:DH] * e
                den = den + src[r0:r0 + G, DH + 1:DH + 2] * e
            ctx32 = jnp.concatenate(
                [num / den, jnp.zeros((S - G, DH), jnp.float32)], axis=0)
            ctx_bh = jnp.where(row_lt_g & is0, ctx32, ctx_bh)
            ctx_parts.append(ctx_bh)
        ctx_b = jnp.concatenate(ctx_parts, axis=1)
        out_ref[b] = jnp.dot(ctx_b, wo_ref[...],
                             preferred_element_type=jnp.float32)

    dA_ltop.wait_send()
    dA_rbot.wait_send()
    dA_r32.wait_send()
    dB.wait_send()


def kernel(x, Wq, K_ext, V_ext, Wo):
    return pl.pallas_call(
        _body,
        out_shape=jax.ShapeDtypeStruct(x.shape, jnp.float32),
        in_specs=[pl.BlockSpec(memory_space=pltpu.VMEM)] * 5,
        out_specs=pl.BlockSpec(memory_space=pltpu.VMEM),
        scratch_shapes=[
            pltpu.VMEM((S, KVW), jnp.float32),
            pltpu.VMEM((2, S, KVW), jnp.float32),
            pltpu.VMEM((G, KVW), jnp.float32),
            pltpu.VMEM((B * G, HD), jnp.float32),
            pltpu.VMEM((B * HQ * G, 128), jnp.float32),
            pltpu.VMEM((B * HQ * G, 128), jnp.float32),
            pltpu.VMEM((3, B * HQ * G, 128), jnp.float32),
            pltpu.SemaphoreType.DMA((8,)),
            pltpu.SemaphoreType.DMA((8,)),
        ],
        compiler_params=pltpu.CompilerParams(collective_id=0),
    )(x, Wq, K_ext, V_ext, Wo)


# device time: 22083 ns/iter; 1.4739x vs baseline; 1.4739x over previous
import jax
import jax.numpy as jnp
from jax import lax
from jax.experimental import pallas as pl
from jax.experimental.pallas import tpu as pltpu

N_DEV = 4
B = 2
S = 256
HQ = 4
DH = 64
HD = HQ * DH
KVW = 2 * B * HD
NEG = -1e9
HALF = S // 2
G = 32
STW = 128

_sem_signal = getattr(pl, "semaphore_signal", None) or pltpu.semaphore_signal
_sem_wait = getattr(pl, "semaphore_wait", None) or pltpu.semaphore_wait
_DeviceIdType = getattr(pl, "DeviceIdType", None) or pltpu.DeviceIdType


def _body(x_ref, wq_ref, k_ref, v_ref, wo_ref, out_ref,
          kv_s, kv_full, far32, q32_buf, st_own, st_in, ss, rs):
    my_pos = lax.axis_index("i")
    right = lax.rem(my_pos + 1, N_DEV)
    left = lax.rem(my_pos + N_DEV - 1, N_DEV)
    dn = (((1,), (1,)), ((), ()))

    for b in range(B):
        kv_s[:, b * 2 * HD:b * 2 * HD + HD] = k_ref[b].reshape(S, HD)
        kv_s[:, b * 2 * HD + HD:(b + 1) * 2 * HD] = v_ref[b].reshape(S, HD)
    far32[...] = jnp.zeros((G, KVW), jnp.float32)

    barrier_sem = pltpu.get_barrier_semaphore()
    for nbr in (left, right):
        _sem_signal(barrier_sem, inc=1, device_id=(nbr,),
                    device_id_type=_DeviceIdType.MESH)
    _sem_wait(barrier_sem, 2)

    @pl.when(my_pos == 0)
    def _():
        for b in range(B):
            q32_buf[b * G:(b + 1) * G, :] = jnp.dot(
                x_ref[b][0:G, :], wq_ref[...],
                preferred_element_type=jnp.float32) * 0.125
        for dev, sidx in ((1, 4), (2, 5), (3, 6)):
            d = pltpu.make_async_remote_copy(
                src_ref=q32_buf, dst_ref=q32_buf,
                send_sem=ss.at[sidx], recv_sem=rs.at[4],
                device_id=(dev,), device_id_type=_DeviceIdType.MESH)
            d.start()
        dfar = pltpu.make_async_remote_copy(
            src_ref=kv_s.at[pl.ds(0, G)], dst_ref=far32,
            send_sem=ss.at[3], recv_sem=rs.at[3],
            device_id=(2,), device_id_type=_DeviceIdType.MESH)
        dfar.start()

    dA_ltop = pltpu.make_async_remote_copy(
        src_ref=kv_s.at[pl.ds(0, HALF)], dst_ref=kv_full.at[0, pl.ds(0, HALF)],
        send_sem=ss.at[0], recv_sem=rs.at[0],
        device_id=(left,), device_id_type=_DeviceIdType.MESH)
    dA_rbot = pltpu.make_async_remote_copy(
        src_ref=kv_s.at[pl.ds(HALF, HALF)],
        dst_ref=kv_full.at[1, pl.ds(HALF, HALF)],
        send_sem=ss.at[1], recv_sem=rs.at[1],
        device_id=(right,), device_id_type=_DeviceIdType.MESH)
    dA_r32 = pltpu.make_async_remote_copy(
        src_ref=kv_s.at[pl.ds(0, G)], dst_ref=kv_full.at[1, pl.ds(0, G)],
        send_sem=ss.at[2], recv_sem=rs.at[2],
        device_id=(right,), device_id_type=_DeviceIdType.MESH)
    dA_ltop.start()
    dA_rbot.start()
    dA_r32.start()

    q = [jnp.dot(x_ref[b], wq_ref[...],
                 preferred_element_type=jnp.float32) * 0.125
         for b in range(B)]

    @pl.when(my_pos != 0)
    def _():
        recv = pltpu.make_async_remote_copy(
            src_ref=q32_buf, dst_ref=q32_buf,
            send_sem=ss.at[7], recv_sem=rs.at[4],
            device_id=(left,), device_id_type=_DeviceIdType.MESH)
        recv.wait_recv()

    for b in range(B):
        for h in range(HQ):
            c0 = b * 2 * HD + h * DH
            q32 = q32_buf[b * G:(b + 1) * G, h * DH:(h + 1) * DH]
            s = lax.dot_general(q32, kv_s[:, c0:c0 + DH], dn,
                                preferred_element_type=jnp.float32)
            m = jnp.max(s, axis=1, keepdims=True)
            w = jnp.exp(s - m)
            l = jnp.sum(w, axis=1, keepdims=True)
            acc = jnp.dot(w, kv_s[:, c0 + HD:c0 + HD + DH],
                          preferred_element_type=jnp.float32)
            r0 = (b * HQ + h) * G
            st_own[r0:r0 + G, :] = jnp.concatenate(
                [acc, m, l, jnp.zeros((G, STW - DH - 2), jnp.float32)],
                axis=1)

    @pl.when(my_pos != 0)
    def _():
        d = pltpu.make_async_remote_copy(
            src_ref=st_own, dst_ref=st_in.at[my_pos - 1],
            send_sem=ss.at[4], recv_sem=rs.at[my_pos + 4],
            device_id=(0,), device_id_type=_DeviceIdType.MESH)
        d.start()

    dA_ltop.wait_recv()
    dA_rbot.wait_recv()

    qi = my_pos * S + lax.broadcasted_iota(jnp.int32, (S, 1), 0)
    off_own = my_pos * S
    off_R = lax.rem(my_pos + 1, N_DEV) * S
    off_F = lax.rem(my_pos + 2, N_DEV) * S
    off_L = lax.rem(my_pos + 3, N_DEV) * S

    def seg_mask(off, r0, n):
        ki = off + r0 + lax.broadcasted_iota(jnp.int32, (S, n), 1)
        return (jnp.abs(qi - ki) <= 128) | (ki < G) | (qi < G)

    mask1 = jnp.concatenate(
        [seg_mask(off_L, HALF, HALF), seg_mask(off_own, 0, S),
         seg_mask(off_R, 0, HALF)], axis=1)
    mask2 = seg_mask(off_L, 0, G)
    mask3 = seg_mask(off_F, 0, G)

    segs1 = [(kv_full, 1, HALF, HALF), (kv_s, None, 0, S),
             (kv_full, 0, 0, HALF)]

    def seg_k(ref, slot, r0, n, c0):
        blk = ref[...] if slot is None else ref[slot]
        return blk[r0:r0 + n, c0:c0 + DH]

    part = {}
    for b in range(B):
        for h in range(HQ):
            c0 = b * 2 * HD + h * DH
            q_bh = q[b][:, h * DH:(h + 1) * DH]
            s1 = jnp.concatenate(
                [lax.dot_general(q_bh, seg_k(r, sl, r0, n, c0), dn,
                                 preferred_element_type=jnp.float32)
                 for (r, sl, r0, n) in segs1], axis=1)
            s1 = jnp.where(mask1, s1, NEG)
            m1 = jnp.max(s1, axis=1, keepdims=True)
            w1 = jnp.exp(s1 - m1)
            l1 = jnp.sum(w1, axis=1, keepdims=True)
            acc = jnp.zeros((S, DH), jnp.float32)
            col = 0
            for (r, sl, r0, n) in segs1:
                acc = acc + jnp.dot(
                    w1[:, col:col + n], seg_k(r, sl, r0, n, c0 + HD),
                    preferred_element_type=jnp.float32)
                col += n
            part[(b, h)] = (m1, l1, acc)

    def upd(state, s_new, v_keys):
        m, l, acc = state
        mn = jnp.maximum(m, jnp.max(s_new, axis=1, keepdims=True))
        a = jnp.exp(m - mn)
        w = jnp.exp(s_new - mn)
        return (mn, l * a + jnp.sum(w, axis=1, keepdims=True),
                acc * a + jnp.dot(w, v_keys,
                                  preferred_element_type=jnp.float32))

    dA_r32.wait_recv()
    for b in range(B):
        for h in range(HQ):
            c0 = b * 2 * HD + h * DH
            q_bh = q[b][:, h * DH:(h + 1) * DH]
            s2 = lax.dot_general(q_bh, kv_full[1][0:G, c0:c0 + DH], dn,
                                 preferred_element_type=jnp.float32)
            s2 = jnp.where(mask2, s2, NEG)
            part[(b, h)] = upd(part[(b, h)], s2,
                               kv_full[1][0:G, c0 + HD:c0 + HD + DH])

    @pl.when(my_pos == 2)
    def _():
        dfar_recv = pltpu.make_async_remote_copy(
            src_ref=kv_s.at[pl.ds(0, G)], dst_ref=far32,
            send_sem=ss.at[7], recv_sem=rs.at[3],
            device_id=(0,), device_id_type=_DeviceIdType.MESH)
        dfar_recv.wait_recv()

    for b in range(B):
        for h in range(HQ):
            c0 = b * 2 * HD + h * DH
            q_bh = q[b][:, h * DH:(h + 1) * DH]
            s3 = lax.dot_general(q_bh, far32[:, c0:c0 + DH], dn,
                                 preferred_element_type=jnp.float32)
            s3 = jnp.where(mask3, s3, NEG)
            part[(b, h)] = upd(part[(b, h)], s3,
                               far32[:, c0 + HD:c0 + HD + DH])

    @pl.when(my_pos == 0)
    def _():
        for j in range(3):
            w = pltpu.make_async_remote_copy(
                src_ref=st_own, dst_ref=st_in.at[j],
                send_sem=ss.at[7], recv_sem=rs.at[j + 5],
                device_id=(right,), device_id_type=_DeviceIdType.MESH)
            w.wait_recv()

    is0 = my_pos == 0
    row_lt_g = lax.broadcasted_iota(jnp.int32, (S, 1), 0) < G
    for b in range(B):
        ctx_parts = []
        for h in range(HQ):
            m1, l1, acc = part[(b, h)]
            ctx_bh = acc / l1
            r0 = (b * HQ + h) * G
            vals = [st_own[r0:r0 + G, :]] + [st_in[j, r0:r0 + G, :]
                                             for j in range(3)]
            ms = [v[:, DH:DH + 1] for v in vals]
            mstar = ms[0]
            for mj in ms[1:]:
                mstar = jnp.maximum(mstar, mj)
            num = jnp.zeros((G, DH), jnp.float32)
            den = jnp.zeros((G, 1), jnp.float32)
            for v, mj in zip(vals, ms):
                e = jnp.exp(mj - mstar)
                num = num + v[:, 0:DH] * e
                den = den + v[:, DH + 1:DH + 2] * e
            ctx32 = jnp.concatenate(
                [num / den, jnp.zeros((S - G, DH), jnp.float32)], axis=0)
            ctx_bh = jnp.where(row_lt_g & is0, ctx32, ctx_bh)
            ctx_parts.append(ctx_bh)
        ctx_b = jnp.concatenate(ctx_parts, axis=1)
        out_ref[b] = jnp.dot(ctx_b, wo_ref[...],
                             preferred_element_type=jnp.float32)

    dA_ltop.wait_send()
    dA_rbot.wait_send()
    dA_r32.wait_send()

    @pl.when(my_pos == 0)
    def _():
        for sidx in (3, 4, 5, 6):
            d = pltpu.make_async_remote_copy(
                src_ref=q32_buf if sidx != 3 else kv_s.at[pl.ds(0, G)],
                dst_ref=q32_buf if sidx != 3 else far32,
                send_sem=ss.at[sidx], recv_sem=rs.at[7],
                device_id=(right,), device_id_type=_DeviceIdType.MESH)
            d.wait_send()

    @pl.when(my_pos != 0)
    def _():
        d = pltpu.make_async_remote_copy(
            src_ref=st_own, dst_ref=st_in.at[0],
            send_sem=ss.at[4], recv_sem=rs.at[7],
            device_id=(0,), device_id_type=_DeviceIdType.MESH)
        d.wait_send()


def kernel(x, Wq, K_ext, V_ext, Wo):
    return pl.pallas_call(
        _body,
        out_shape=jax.ShapeDtypeStruct(x.shape, jnp.float32),
        in_specs=[pl.BlockSpec(memory_space=pltpu.VMEM)] * 5,
        out_specs=pl.BlockSpec(memory_space=pltpu.VMEM),
        scratch_shapes=[
            pltpu.VMEM((S, KVW), jnp.float32),
            pltpu.VMEM((2, S, KVW), jnp.float32),
            pltpu.VMEM((G, KVW), jnp.float32),
            pltpu.VMEM((B * G, HD), jnp.float32),
            pltpu.VMEM((B * HQ * G, STW), jnp.float32),
            pltpu.VMEM((3, B * HQ * G, STW), jnp.float32),
            pltpu.SemaphoreType.DMA((8,)),
            pltpu.SemaphoreType.DMA((8,)),
        ],
        compiler_params=pltpu.CompilerParams(collective_id=0),
    )(x, Wq, K_ext, V_ext, Wo)


# device time: 20845 ns/iter; 1.5615x vs baseline; 1.0594x over previous
import jax
import jax.numpy as jnp
from jax import lax
from jax.experimental import pallas as pl
from jax.experimental.pallas import tpu as pltpu

N_DEV = 4
B = 2
S = 256
HQ = 4
DH = 64
HD = HQ * DH
KVW = 2 * B * HD
NEG = -1e9
HALF = S // 2
G = 32
STW = 128

_sem_signal = getattr(pl, "semaphore_signal", None) or pltpu.semaphore_signal
_sem_wait = getattr(pl, "semaphore_wait", None) or pltpu.semaphore_wait
_DeviceIdType = getattr(pl, "DeviceIdType", None) or pltpu.DeviceIdType


def _body(x_ref, wq_ref, k_ref, v_ref, wo_ref, out_ref,
          kv_s, kv_full, far32, q32_buf, st_own, st_in, ss, rs):
    my_pos = lax.axis_index("i")
    right = lax.rem(my_pos + 1, N_DEV)
    left = lax.rem(my_pos + N_DEV - 1, N_DEV)
    dn = (((1,), (1,)), ((), ()))

    for b in range(B):
        kv_s[:, b * 2 * HD:b * 2 * HD + HD] = k_ref[b].reshape(S, HD)
        kv_s[:, b * 2 * HD + HD:(b + 1) * 2 * HD] = v_ref[b].reshape(S, HD)
    far32[...] = jnp.zeros((G, KVW), jnp.float32)

    barrier_sem = pltpu.get_barrier_semaphore()
    for nbr in (left, right):
        _sem_signal(barrier_sem, inc=1, device_id=(nbr,),
                    device_id_type=_DeviceIdType.MESH)
    _sem_wait(barrier_sem, 2)

    @pl.when(my_pos == 0)
    def _():
        for b in range(B):
            q32_buf[b * G:(b + 1) * G, :] = jnp.dot(
                x_ref[b][0:G, :], wq_ref[...],
                preferred_element_type=jnp.float32) * 0.125
        for dev, sidx in ((1, 4), (2, 5), (3, 6)):
            d = pltpu.make_async_remote_copy(
                src_ref=q32_buf, dst_ref=q32_buf,
                send_sem=ss.at[sidx], recv_sem=rs.at[4],
                device_id=(dev,), device_id_type=_DeviceIdType.MESH)
            d.start()
        dfar = pltpu.make_async_remote_copy(
            src_ref=kv_s.at[pl.ds(0, G)], dst_ref=far32,
            send_sem=ss.at[3], recv_sem=rs.at[3],
            device_id=(2,), device_id_type=_DeviceIdType.MESH)
        dfar.start()

    dA_ltop = pltpu.make_async_remote_copy(
        src_ref=kv_s.at[pl.ds(0, HALF)], dst_ref=kv_full.at[0, pl.ds(0, HALF)],
        send_sem=ss.at[0], recv_sem=rs.at[0],
        device_id=(left,), device_id_type=_DeviceIdType.MESH)
    dA_rbot = pltpu.make_async_remote_copy(
        src_ref=kv_s.at[pl.ds(HALF, HALF)],
        dst_ref=kv_full.at[1, pl.ds(HALF, HALF)],
        send_sem=ss.at[1], recv_sem=rs.at[1],
        device_id=(right,), device_id_type=_DeviceIdType.MESH)
    dA_r32 = pltpu.make_async_remote_copy(
        src_ref=kv_s.at[pl.ds(0, G)], dst_ref=kv_full.at[1, pl.ds(0, G)],
        send_sem=ss.at[2], recv_sem=rs.at[2],
        device_id=(right,), device_id_type=_DeviceIdType.MESH)
    dA_ltop.start()
    dA_rbot.start()
    dA_r32.start()

    q = [jnp.dot(x_ref[b], wq_ref[...],
                 preferred_element_type=jnp.float32) * 0.125
         for b in range(B)]

    @pl.when(my_pos != 0)
    def _():
        recv = pltpu.make_async_remote_copy(
            src_ref=q32_buf, dst_ref=q32_buf,
            send_sem=ss.at[7], recv_sem=rs.at[4],
            device_id=(left,), device_id_type=_DeviceIdType.MESH)
        recv.wait_recv()

    for b in range(B):
        for h in range(HQ):
            c0 = b * 2 * HD + h * DH
            q32 = q32_buf[b * G:(b + 1) * G, h * DH:(h + 1) * DH]
            s = lax.dot_general(q32, kv_s[:, c0:c0 + DH], dn,
                                preferred_element_type=jnp.float32)
            m = jnp.max(s, axis=1, keepdims=True)
            w = jnp.exp(s - m)
            l = jnp.sum(w, axis=1, keepdims=True)
            acc = jnp.dot(w, kv_s[:, c0 + HD:c0 + HD + DH],
                          preferred_element_type=jnp.float32)
            r0 = (b * HQ + h) * G
            st_own[r0:r0 + G, :] = jnp.concatenate(
                [acc, m, l, jnp.zeros((G, STW - DH - 2), jnp.float32)],
                axis=1)

    @pl.when(my_pos != 0)
    def _():
        d = pltpu.make_async_remote_copy(
            src_ref=st_own, dst_ref=st_in.at[my_pos - 1],
            send_sem=ss.at[4], recv_sem=rs.at[my_pos + 4],
            device_id=(0,), device_id_type=_DeviceIdType.MESH)
        d.start()

    dA_ltop.wait_recv()
    dA_rbot.wait_recv()

    qi = my_pos * S + lax.broadcasted_iota(jnp.int32, (S, 1), 0)
    off_own = my_pos * S
    off_R = lax.rem(my_pos + 1, N_DEV) * S
    off_F = lax.rem(my_pos + 2, N_DEV) * S
    off_L = lax.rem(my_pos + 3, N_DEV) * S

    def seg_mask(off, r0, n):
        ki = off + r0 + lax.broadcasted_iota(jnp.int32, (S, n), 1)
        return (jnp.abs(qi - ki) <= 128) | (ki < G) | (qi < G)

    mask1 = jnp.concatenate(
        [seg_mask(off_L, HALF, HALF), seg_mask(off_own, 0, S),
         seg_mask(off_R, 0, HALF)], axis=1)

    segs1 = [(kv_full, 1, HALF, HALF), (kv_s, None, 0, S),
             (kv_full, 0, 0, HALF)]

    def seg_k(ref, slot, r0, n, c0):
        blk = ref[...] if slot is None else ref[slot]
        return blk[r0:r0 + n, c0:c0 + DH]

    part = {}
    for b in range(B):
        for h in range(HQ):
            c0 = b * 2 * HD + h * DH
            q_bh = q[b][:, h * DH:(h + 1) * DH]
            s1 = jnp.concatenate(
                [lax.dot_general(q_bh, seg_k(r, sl, r0, n, c0), dn,
                                 preferred_element_type=jnp.float32)
                 for (r, sl, r0, n) in segs1], axis=1)
            s1 = jnp.where(mask1, s1, NEG)
            m1 = jnp.max(s1, axis=1, keepdims=True)
            w1 = jnp.exp(s1 - m1)
            l1 = jnp.sum(w1, axis=1, keepdims=True)
            acc = jnp.zeros((S, DH), jnp.float32)
            col = 0
            for (r, sl, r0, n) in segs1:
                acc = acc + jnp.dot(
                    w1[:, col:col + n], seg_k(r, sl, r0, n, c0 + HD),
                    preferred_element_type=jnp.float32)
                col += n
            part[(b, h)] = (m1, l1, acc)

    def upd(state, s_new, v_keys):
        m, l, acc = state
        mn = jnp.maximum(m, jnp.max(s_new, axis=1, keepdims=True))
        a = jnp.exp(m - mn)
        w = jnp.exp(s_new - mn)
        return (mn, l * a + jnp.sum(w, axis=1, keepdims=True),
                acc * a + jnp.dot(w, v_keys,
                                  preferred_element_type=jnp.float32))

    dA_r32.wait_recv()

    @pl.when(my_pos == 2)
    def _():
        dfar_recv = pltpu.make_async_remote_copy(
            src_ref=kv_s.at[pl.ds(0, G)], dst_ref=far32,
            send_sem=ss.at[7], recv_sem=rs.at[3],
            device_id=(0,), device_id_type=_DeviceIdType.MESH)
        dfar_recv.wait_recv()

    col23 = lax.broadcasted_iota(jnp.int32, (S, 2 * G), 1)
    ki23 = jnp.where(col23 < G, off_L + col23, off_F + col23 - G)
    mask23 = (jnp.abs(qi - ki23) <= 128) | (ki23 < G) | (qi < G)
    for b in range(B):
        for h in range(HQ):
            c0 = b * 2 * HD + h * DH
            q_bh = q[b][:, h * DH:(h + 1) * DH]
            k23 = jnp.concatenate(
                [kv_full[1][0:G, c0:c0 + DH], far32[:, c0:c0 + DH]], axis=0)
            v23 = jnp.concatenate(
                [kv_full[1][0:G, c0 + HD:c0 + HD + DH],
                 far32[:, c0 + HD:c0 + HD + DH]], axis=0)
            s23 = lax.dot_general(q_bh, k23, dn,
                                  preferred_element_type=jnp.float32)
            s23 = jnp.where(mask23, s23, NEG)
            part[(b, h)] = upd(part[(b, h)], s23, v23)

    @pl.when(my_pos == 0)
    def _():
        for j in range(3):
            w = pltpu.make_async_remote_copy(
                src_ref=st_own, dst_ref=st_in.at[j],
                send_sem=ss.at[7], recv_sem=rs.at[j + 5],
                device_id=(right,), device_id_type=_DeviceIdType.MESH)
            w.wait_recv()

    vals = [st_own[...]] + [st_in[j] for j in range(3)]
    ms = [v[:, DH:DH + 1] for v in vals]
    mstar = ms[0]
    for mj in ms[1:]:
        mstar = jnp.maximum(mstar, mj)
    num = jnp.zeros((B * HQ * G, DH), jnp.float32)
    den = jnp.zeros((B * HQ * G, 1), jnp.float32)
    for v, mj in zip(vals, ms):
        e = jnp.exp(mj - mstar)
        num = num + v[:, 0:DH] * e
        den = den + v[:, DH + 1:DH + 2] * e
    ctx32_all = num / den

    is0 = my_pos == 0
    row_lt_g = lax.broadcasted_iota(jnp.int32, (S, 1), 0) < G
    for b in range(B):
        ctx_parts = []
        for h in range(HQ):
            m1, l1, acc = part[(b, h)]
            ctx_bh = acc / l1
            r0 = (b * HQ + h) * G
            ctx32 = jnp.concatenate(
                [ctx32_all[r0:r0 + G, :],
                 jnp.zeros((S - G, DH), jnp.float32)], axis=0)
            ctx_bh = jnp.where(row_lt_g & is0, ctx32, ctx_bh)
            ctx_parts.append(ctx_bh)
        ctx_b = jnp.concatenate(ctx_parts, axis=1)
        out_ref[b] = jnp.dot(ctx_b, wo_ref[...],
                             preferred_element_type=jnp.float32)

    dA_ltop.wait_send()
    dA_rbot.wait_send()
    dA_r32.wait_send()

    @pl.when(my_pos == 0)
    def _():
        for sidx in (3, 4, 5, 6):
            d = pltpu.make_async_remote_copy(
                src_ref=q32_buf if sidx != 3 else kv_s.at[pl.ds(0, G)],
                dst_ref=q32_buf if sidx != 3 else far32,
                send_sem=ss.at[sidx], recv_sem=rs.at[7],
                device_id=(right,), device_id_type=_DeviceIdType.MESH)
            d.wait_send()

    @pl.when(my_pos != 0)
    def _():
        d = pltpu.make_async_remote_copy(
            src_ref=st_own, dst_ref=st_in.at[0],
            send_sem=ss.at[4], recv_sem=rs.at[7],
            device_id=(0,), device_id_type=_DeviceIdType.MESH)
        d.wait_send()


def kernel(x, Wq, K_ext, V_ext, Wo):
    return pl.pallas_call(
        _body,
        out_shape=jax.ShapeDtypeStruct(x.shape, jnp.float32),
        in_specs=[pl.BlockSpec(memory_space=pltpu.VMEM)] * 5,
        out_specs=pl.BlockSpec(memory_space=pltpu.VMEM),
        scratch_shapes=[
            pltpu.VMEM((S, KVW), jnp.float32),
            pltpu.VMEM((2, S, KVW), jnp.float32),
            pltpu.VMEM((G, KVW), jnp.float32),
            pltpu.VMEM((B * G, HD), jnp.float32),
            pltpu.VMEM((B * HQ * G, STW), jnp.float32),
            pltpu.VMEM((3, B * HQ * G, STW), jnp.float32),
            pltpu.SemaphoreType.DMA((8,)),
            pltpu.SemaphoreType.DMA((8,)),
        ],
        compiler_params=pltpu.CompilerParams(collective_id=0),
    )(x, Wq, K_ext, V_ext, Wo)


# device time: 19361 ns/iter; 1.6812x vs baseline; 1.0766x over previous
import jax
import jax.numpy as jnp
from jax import lax
from jax.experimental import pallas as pl
from jax.experimental.pallas import tpu as pltpu

N_DEV = 4
B = 2
S = 256
HQ = 4
DH = 64
HD = HQ * DH
KVW = 2 * B * HD
NEG = -1e9
HALF = S // 2
G = 32
STW = 128

_sem_signal = getattr(pl, "semaphore_signal", None) or pltpu.semaphore_signal
_sem_wait = getattr(pl, "semaphore_wait", None) or pltpu.semaphore_wait
_DeviceIdType = getattr(pl, "DeviceIdType", None) or pltpu.DeviceIdType


def _body(x_ref, wq_ref, k_ref, v_ref, wo_ref, out_ref,
          kv_s, kv_full, far32, q32_buf, st_own, st_in, ss, rs):
    my_pos = lax.axis_index("i")
    right = lax.rem(my_pos + 1, N_DEV)
    left = lax.rem(my_pos + N_DEV - 1, N_DEV)
    dn = (((1,), (1,)), ((), ()))

    for b in range(B):
        kv_s[:, b * 2 * HD:b * 2 * HD + HD] = k_ref[b].reshape(S, HD)
        kv_s[:, b * 2 * HD + HD:(b + 1) * 2 * HD] = v_ref[b].reshape(S, HD)
    far32[...] = jnp.zeros((G, KVW), jnp.float32)

    barrier_sem = pltpu.get_barrier_semaphore()
    for nbr in (left, right):
        _sem_signal(barrier_sem, inc=1, device_id=(nbr,),
                    device_id_type=_DeviceIdType.MESH)
    _sem_wait(barrier_sem, 2)

    @pl.when(my_pos == 0)
    def _():
        for b in range(B):
            q32_buf[b * G:(b + 1) * G, :] = jnp.dot(
                x_ref[b][0:G, :], wq_ref[...],
                preferred_element_type=jnp.float32) * 0.125
        for dev, sidx in ((1, 4), (2, 5), (3, 6)):
            d = pltpu.make_async_remote_copy(
                src_ref=q32_buf, dst_ref=q32_buf,
                send_sem=ss.at[sidx], recv_sem=rs.at[4],
                device_id=(dev,), device_id_type=_DeviceIdType.MESH)
            d.start()
        dfar = pltpu.make_async_remote_copy(
            src_ref=kv_s.at[pl.ds(0, G)], dst_ref=far32,
            send_sem=ss.at[3], recv_sem=rs.at[3],
            device_id=(2,), device_id_type=_DeviceIdType.MESH)
        dfar.start()

    dA_ltop = pltpu.make_async_remote_copy(
        src_ref=kv_s.at[pl.ds(0, HALF)], dst_ref=kv_full.at[0, pl.ds(0, HALF)],
        send_sem=ss.at[0], recv_sem=rs.at[0],
        device_id=(left,), device_id_type=_DeviceIdType.MESH)
    dA_rbot = pltpu.make_async_remote_copy(
        src_ref=kv_s.at[pl.ds(HALF, HALF)],
        dst_ref=kv_full.at[1, pl.ds(HALF, HALF)],
        send_sem=ss.at[1], recv_sem=rs.at[1],
        device_id=(right,), device_id_type=_DeviceIdType.MESH)
    dA_r32 = pltpu.make_async_remote_copy(
        src_ref=kv_s.at[pl.ds(0, G)], dst_ref=kv_full.at[1, pl.ds(0, G)],
        send_sem=ss.at[2], recv_sem=rs.at[2],
        device_id=(right,), device_id_type=_DeviceIdType.MESH)
    dA_ltop.start()
    dA_rbot.start()
    dA_r32.start()

    q = [jnp.dot(x_ref[b], wq_ref[...],
                 preferred_element_type=jnp.float32) * 0.125
         for b in range(B)]

    @pl.when(my_pos != 0)
    def _():
        recv = pltpu.make_async_remote_copy(
            src_ref=q32_buf, dst_ref=q32_buf,
            send_sem=ss.at[7], recv_sem=rs.at[4],
            device_id=(left,), device_id_type=_DeviceIdType.MESH)
        recv.wait_recv()

    for b in range(B):
        for h in range(HQ):
            c0 = b * 2 * HD + h * DH
            q32 = q32_buf[b * G:(b + 1) * G, h * DH:(h + 1) * DH]
            s = lax.dot_general(q32, kv_s[:, c0:c0 + DH], dn,
                                preferred_element_type=jnp.float32)
            m = jnp.max(s, axis=1, keepdims=True)
            w = jnp.exp(s - m)
            l = jnp.sum(w, axis=1, keepdims=True)
            acc = jnp.dot(w, kv_s[:, c0 + HD:c0 + HD + DH],
                          preferred_element_type=jnp.float32)
            r0 = (b * HQ + h) * G
            st_own[r0:r0 + G, :] = jnp.concatenate(
                [acc, m, l, jnp.zeros((G, STW - DH - 2), jnp.float32)],
                axis=1)

    @pl.when(my_pos != 0)
    def _():
        d = pltpu.make_async_remote_copy(
            src_ref=st_own, dst_ref=st_in.at[my_pos - 1],
            send_sem=ss.at[4], recv_sem=rs.at[my_pos + 4],
            device_id=(0,), device_id_type=_DeviceIdType.MESH)
        d.start()

    qi = my_pos * S + lax.broadcasted_iota(jnp.int32, (S, 1), 0)
    off_own = my_pos * S
    off_R = lax.rem(my_pos + 1, N_DEV) * S
    off_F = lax.rem(my_pos + 2, N_DEV) * S
    off_L = lax.rem(my_pos + 3, N_DEV) * S

    def seg_mask(off, r0, n):
        ki = off + r0 + lax.broadcasted_iota(jnp.int32, (S, n), 1)
        return (jnp.abs(qi - ki) <= 128) | (ki < G) | (qi < G)

    mask_own = seg_mask(off_own, 0, S)
    part = {}
    for b in range(B):
        for h in range(HQ):
            c0 = b * 2 * HD + h * DH
            q_bh = q[b][:, h * DH:(h + 1) * DH]
            s0 = lax.dot_general(q_bh, kv_s[:, c0:c0 + DH], dn,
                                 preferred_element_type=jnp.float32)
            s0 = jnp.where(mask_own, s0, NEG)
            m0 = jnp.max(s0, axis=1, keepdims=True)
            w0 = jnp.exp(s0 - m0)
            l0 = jnp.sum(w0, axis=1, keepdims=True)
            acc = jnp.dot(w0, kv_s[:, c0 + HD:c0 + HD + DH],
                          preferred_element_type=jnp.float32)
            part[(b, h)] = (m0, l0, acc)

    dA_ltop.wait_recv()
    dA_rbot.wait_recv()
    colLR = lax.broadcasted_iota(jnp.int32, (S, 2 * HALF), 1)
    kiLR = jnp.where(colLR < HALF, off_L + HALF + colLR,
                     off_R + colLR - HALF)
    maskLR = (jnp.abs(qi - kiLR) <= 128) | (kiLR < G) | (qi < G)
    for b in range(B):
        for h in range(HQ):
            c0 = b * 2 * HD + h * DH
            q_bh = q[b][:, h * DH:(h + 1) * DH]
            sLR = jnp.concatenate(
                [lax.dot_general(q_bh, kv_full[1][HALF:, c0:c0 + DH], dn,
                                 preferred_element_type=jnp.float32),
                 lax.dot_general(q_bh, kv_full[0][0:HALF, c0:c0 + DH], dn,
                                 preferred_element_type=jnp.float32)],
                axis=1)
            sLR = jnp.where(maskLR, sLR, NEG)
            m, l, acc = part[(b, h)]
            mn = jnp.maximum(m, jnp.max(sLR, axis=1, keepdims=True))
            a = jnp.exp(m - mn)
            w = jnp.exp(sLR - mn)
            l = l * a + jnp.sum(w, axis=1, keepdims=True)
            acc = (acc * a
                   + jnp.dot(w[:, 0:HALF],
                             kv_full[1][HALF:, c0 + HD:c0 + HD + DH],
                             preferred_element_type=jnp.float32)
                   + jnp.dot(w[:, HALF:],
                             kv_full[0][0:HALF, c0 + HD:c0 + HD + DH],
                             preferred_element_type=jnp.float32))
            part[(b, h)] = (mn, l, acc)

    def upd(state, s_new, v_keys):
        m, l, acc = state
        mn = jnp.maximum(m, jnp.max(s_new, axis=1, keepdims=True))
        a = jnp.exp(m - mn)
        w = jnp.exp(s_new - mn)
        return (mn, l * a + jnp.sum(w, axis=1, keepdims=True),
                acc * a + jnp.dot(w, v_keys,
                                  preferred_element_type=jnp.float32))

    dA_r32.wait_recv()

    @pl.when(my_pos == 2)
    def _():
        dfar_recv = pltpu.make_async_remote_copy(
            src_ref=kv_s.at[pl.ds(0, G)], dst_ref=far32,
            send_sem=ss.at[7], recv_sem=rs.at[3],
            device_id=(0,), device_id_type=_DeviceIdType.MESH)
        dfar_recv.wait_recv()

    col23 = lax.broadcasted_iota(jnp.int32, (S, 2 * G), 1)
    ki23 = jnp.where(col23 < G, off_L + col23, off_F + col23 - G)
    mask23 = (jnp.abs(qi - ki23) <= 128) | (ki23 < G) | (qi < G)
    for b in range(B):
        for h in range(HQ):
            c0 = b * 2 * HD + h * DH
            q_bh = q[b][:, h * DH:(h + 1) * DH]
            k23 = jnp.concatenate(
                [kv_full[1][0:G, c0:c0 + DH], far32[:, c0:c0 + DH]], axis=0)
            v23 = jnp.concatenate(
                [kv_full[1][0:G, c0 + HD:c0 + HD + DH],
                 far32[:, c0 + HD:c0 + HD + DH]], axis=0)
            s23 = lax.dot_general(q_bh, k23, dn,
                                  preferred_element_type=jnp.float32)
            s23 = jnp.where(mask23, s23, NEG)
            part[(b, h)] = upd(part[(b, h)], s23, v23)

    @pl.when(my_pos == 0)
    def _():
        for j in range(3):
            w = pltpu.make_async_remote_copy(
                src_ref=st_own, dst_ref=st_in.at[j],
                send_sem=ss.at[7], recv_sem=rs.at[j + 5],
                device_id=(right,), device_id_type=_DeviceIdType.MESH)
            w.wait_recv()

    vals = [st_own[...]] + [st_in[j] for j in range(3)]
    ms = [v[:, DH:DH + 1] for v in vals]
    mstar = ms[0]
    for mj in ms[1:]:
        mstar = jnp.maximum(mstar, mj)
    num = jnp.zeros((B * HQ * G, DH), jnp.float32)
    den = jnp.zeros((B * HQ * G, 1), jnp.float32)
    for v, mj in zip(vals, ms):
        e = jnp.exp(mj - mstar)
        num = num + v[:, 0:DH] * e
        den = den + v[:, DH + 1:DH + 2] * e
    ctx32_all = num / den

    is0 = my_pos == 0
    row_lt_g = lax.broadcasted_iota(jnp.int32, (S, 1), 0) < G
    for b in range(B):
        ctx_parts = []
        for h in range(HQ):
            m1, l1, acc = part[(b, h)]
            ctx_bh = acc / l1
            r0 = (b * HQ + h) * G
            ctx32 = jnp.concatenate(
                [ctx32_all[r0:r0 + G, :],
                 jnp.zeros((S - G, DH), jnp.float32)], axis=0)
            ctx_bh = jnp.where(row_lt_g & is0, ctx32, ctx_bh)
            ctx_parts.append(ctx_bh)
        ctx_b = jnp.concatenate(ctx_parts, axis=1)
        out_ref[b] = jnp.dot(ctx_b, wo_ref[...],
                             preferred_element_type=jnp.float32)

    dA_ltop.wait_send()
    dA_rbot.wait_send()
    dA_r32.wait_send()

    @pl.when(my_pos == 0)
    def _():
        for sidx in (3, 4, 5, 6):
            d = pltpu.make_async_remote_copy(
                src_ref=q32_buf if sidx != 3 else kv_s.at[pl.ds(0, G)],
                dst_ref=q32_buf if sidx != 3 else far32,
                send_sem=ss.at[sidx], recv_sem=rs.at[7],
                device_id=(right,), device_id_type=_DeviceIdType.MESH)
            d.wait_send()

    @pl.when(my_pos != 0)
    def _():
        d = pltpu.make_async_remote_copy(
            src_ref=st_own, dst_ref=st_in.at[0],
            send_sem=ss.at[4], recv_sem=rs.at[7],
            device_id=(0,), device_id_type=_DeviceIdType.MESH)
        d.wait_send()


def kernel(x, Wq, K_ext, V_ext, Wo):
    return pl.pallas_call(
        _body,
        out_shape=jax.ShapeDtypeStruct(x.shape, jnp.float32),
        in_specs=[pl.BlockSpec(memory_space=pltpu.VMEM)] * 5,
        out_specs=pl.BlockSpec(memory_space=pltpu.VMEM),
        scratch_shapes=[
            pltpu.VMEM((S, KVW), jnp.float32),
            pltpu.VMEM((2, S, KVW), jnp.float32),
            pltpu.VMEM((G, KVW), jnp.float32),
            pltpu.VMEM((B * G, HD), jnp.float32),
            pltpu.VMEM((B * HQ * G, STW), jnp.float32),
            pltpu.VMEM((3, B * HQ * G, STW), jnp.float32),
            pltpu.SemaphoreType.DMA((8,)),
            pltpu.SemaphoreType.DMA((8,)),
        ],
        compiler_params=pltpu.CompilerParams(collective_id=0),
    )(x, Wq, K_ext, V_ext, Wo)
